# baseline (speedup 1.0000x reference)
import functools

import jax
import jax.numpy as jnp
from jax import lax
from jax.experimental import pallas as pl
from jax.experimental.pallas import tpu as pltpu
from jax.experimental.pallas import tpu_sc as plsc

_NUM_CORES = 2
_NUM_SUBCORES = 16
_NUM_WORKERS = _NUM_CORES * _NUM_SUBCORES
_L = 16
_NCH = 4
_RB = 4


@functools.lru_cache(maxsize=None)
def _make_sc(V, D, B):
    assert B % (_NUM_WORKERS * _NCH) == 0 and D % _L == 0
    b_per_w = B // _NUM_WORKERS
    rows_c = b_per_w // _NCH
    dch = D // _L
    mesh = plsc.VectorSubcoreMesh(core_axis_name="c", subcore_axis_name="s")

    @functools.partial(
        pl.kernel,
        mesh=mesh,
        out_type=jax.ShapeDtypeStruct((B, D), jnp.float32),
        scratch_types=[
            pltpu.VMEM((V, D), jnp.float32),
            pltpu.VMEM((b_per_w, _L), jnp.int32),
            pltpu.VMEM((rows_c, D), jnp.float32),
            pltpu.VMEM((rows_c, D), jnp.float32),
            pltpu.SemaphoreType.DMA,
            pltpu.SemaphoreType.DMA,
        ],
    )
    def k(table_hbm, idxrep_hbm, out_hbm, tab_v, idxr_v, buf0, buf1,
          sw0, sw1):
        wid = lax.axis_index("s") * _NUM_CORES + lax.axis_index("c")
        base = wid * b_per_w
        bufs = (buf0, buf1)
        sw = (sw0, sw1)
        pltpu.sync_copy(table_hbm, tab_v)
        pltpu.sync_copy(idxrep_hbm.at[pl.ds(base, b_per_w)], idxr_v)

        writes = [None, None]
        for c in range(_NCH):
            b = c & 1
            if writes[b] is not None:
                writes[b].wait()
            buf = bufs[b]
            for rb in range(rows_c // _RB):
                r0 = rb * _RB
                oh = []
                for j in range(_RB):
                    rv = idxr_v[c * rows_c + r0 + j, :]
                    oh.append([rv == v for v in range(V - 1)])

                def body(i, carry, oh=oh, buf=buf, r0=r0):
                    sl = pl.ds(i * _L, _L)
                    t = [tab_v[v, sl] for v in range(V)]
                    for j in range(_RB):
                        col = t[V - 1]
                        for v in range(V - 2, -1, -1):
                            col = jnp.where(oh[j][v], t[v], col)
                        buf[r0 + j, sl] = col
                    return carry

                lax.fori_loop(0, dch, body, jnp.int32(0), unroll=4)
            writes[b] = pltpu.async_copy(
                buf, out_hbm.at[pl.ds(base + c * rows_c, rows_c)], sw[b])
        for w in writes:
            if w is not None:
                w.wait()

    return k


def kernel(disease_embeds, disease_indices):
    V, D = disease_embeds.shape
    (B,) = disease_indices.shape
    idx_rep = jnp.broadcast_to(
        disease_indices.astype(jnp.int32)[:, None], (B, _L))
    return _make_sc(V, D, B)(disease_embeds, idx_rep)


# speedup vs baseline: 1.2925x; 1.2925x over previous
import functools

import jax
import jax.numpy as jnp
from jax import lax
from jax.experimental import pallas as pl
from jax.experimental.pallas import tpu as pltpu
from jax.experimental.pallas import tpu_sc as plsc

_NUM_CORES = 2
_NUM_SUBCORES = 16
_NUM_WORKERS = _NUM_CORES * _NUM_SUBCORES
_L = 16


@functools.lru_cache(maxsize=None)
def _make_sc(V, D, B):
    assert B % (_NUM_WORKERS * _L) == 0
    b_per_w = B // _NUM_WORKERS
    mesh = plsc.VectorSubcoreMesh(core_axis_name="c", subcore_axis_name="s")

    @functools.partial(
        pl.kernel,
        mesh=mesh,
        out_type=jax.ShapeDtypeStruct((B, D), jnp.float32),
        scratch_types=[
            pltpu.VMEM((V, D), jnp.float32),
            pltpu.VMEM((b_per_w,), jnp.int32),
            pltpu.SemaphoreType.DMA,
        ],
    )
    def k(table_hbm, idx_hbm, out_hbm, tab_v, idx_v, sem):
        wid = lax.axis_index("s") * _NUM_CORES + lax.axis_index("c")
        base = wid * b_per_w
        pltpu.sync_copy(table_hbm, tab_v)
        pltpu.sync_copy(idx_hbm.at[pl.ds(base, b_per_w)], idx_v)
        handles = []
        for g in range(b_per_w // _L):
            vec = idx_v[pl.ds(g * _L, _L)]
            for j in range(_L):
                handles.append(pltpu.async_copy(
                    tab_v.at[pl.ds(vec[j], 1)],
                    out_hbm.at[pl.ds(base + g * _L + j, 1)],
                    sem))
        for h in handles:
            h.wait()

    return k


def kernel(disease_embeds, disease_indices):
    V, D = disease_embeds.shape
    (B,) = disease_indices.shape
    idx = disease_indices.astype(jnp.int32)
    return _make_sc(V, D, B)(disease_embeds, idx)


# speedup vs baseline: 1.5088x; 1.1673x over previous
import functools

import jax
import jax.numpy as jnp
from jax import lax
from jax.experimental import pallas as pl
from jax.experimental.pallas import tpu as pltpu
from jax.experimental.pallas import tpu_sc as plsc

_NUM_CORES = 2
_NUM_SUBCORES = 16
_NUM_WORKERS = _NUM_CORES * _NUM_SUBCORES
_L = 16


@functools.lru_cache(maxsize=None)
def _make_sc(V, D, B):
    assert B % (_NUM_WORKERS * _L) == 0
    b_per_w = B // _NUM_WORKERS
    mesh = plsc.VectorSubcoreMesh(core_axis_name="c", subcore_axis_name="s")

    @functools.partial(
        pl.kernel,
        mesh=mesh,
        out_type=jax.ShapeDtypeStruct((B, D), jnp.float32),
        scratch_types=[
            pltpu.VMEM_SHARED((8 * V * V, D), jnp.float32),
            pltpu.VMEM((b_per_w,), jnp.int32),
            pltpu.SemaphoreType.DMA,
        ],
    )
    def k(pairs_hbm, idx_hbm, out_hbm, pairs_sh, idx_v, sem):
        sid = lax.axis_index("s")
        wid = sid * _NUM_CORES + lax.axis_index("c")
        base = wid * b_per_w

        @pl.when(sid == 0)
        def _stage():
            pltpu.sync_copy(pairs_hbm, pairs_sh)

        pltpu.sync_copy(idx_hbm.at[pl.ds(base, b_per_w)], idx_v)
        plsc.subcore_barrier()
        handles = []
        for g in range(b_per_w // _L):
            vec = idx_v[pl.ds(g * _L, _L)]
            for j in range(0, _L, 2):
                s = (vec[j] * V + vec[j + 1]) * 8
                handles.append(pltpu.async_copy(
                    pairs_sh.at[pl.ds(s, 2)],
                    out_hbm.at[pl.ds(base + g * _L + j, 2)],
                    sem))
        for h in handles:
            h.wait()

    return k


def kernel(disease_embeds, disease_indices):
    V, D = disease_embeds.shape
    (B,) = disease_indices.shape
    idx = disease_indices.astype(jnp.int32)
    pairs = jnp.stack(
        [jnp.repeat(disease_embeds, V, axis=0),
         jnp.tile(disease_embeds, (V, 1))], axis=1)
    pairs_pad = jnp.pad(pairs, ((0, 0), (0, 6), (0, 0))).reshape(
        8 * V * V, D)
    return _make_sc(V, D, B)(pairs_pad, idx)
